# Initial kernel scaffold; baseline (speedup 1.0000x reference)
import jax
import jax.numpy as jnp
from jax.experimental import pallas as pl


def kernel(x, table):
    raise NotImplementedError("write your pallas kernel here")


# speedup vs baseline: 2.9423x; 2.9423x over previous
import functools
import math

import jax
import jax.numpy as jnp
from jax import lax
from jax.experimental import pallas as pl
from jax.experimental.pallas import tpu as pltpu
from jax.experimental.pallas import tpu_sc as plsc

B, L, D, VOCAB = 1024, 200, 128, 100000
HALF = D // 2
CH = 100
NCH = L // CH
NW = 32
SEQ_PER_W = B // NW


def _trig_body(cos_ref, sin_ref):
    pos = lax.broadcasted_iota(jnp.int32, (L, HALF), 0).astype(jnp.float32)
    fi = lax.broadcasted_iota(jnp.int32, (L, HALF), 1).astype(jnp.float32)
    ang = pos * jnp.exp(fi * (-math.log(10000.0) / D))
    cos_ref[...] = jnp.cos(ang)
    sin_ref[...] = jnp.sin(ang)


def _make_tables():
    return pl.pallas_call(
        _trig_body,
        out_shape=(
            jax.ShapeDtypeStruct((L, HALF), jnp.float32),
            jax.ShapeDtypeStruct((L, HALF), jnp.float32),
        ),
    )()


@functools.partial(
    pl.kernel,
    mesh=plsc.VectorSubcoreMesh(core_axis_name="c", subcore_axis_name="s"),
    out_type=jax.ShapeDtypeStruct((B, NCH, CH, D), jnp.float32),
    scratch_types=[
        pltpu.VMEM((NCH, CH), jnp.int32),
        pltpu.VMEM((NCH, CH, D), jnp.float32),
        pltpu.VMEM((NCH, CH, D), jnp.float32),
        pltpu.VMEM((L, HALF), jnp.float32),
        pltpu.VMEM((L, HALF), jnp.float32),
        pltpu.SemaphoreType.DMA,
    ],
)
def _sc_rope_gather(x_hbm, table_hbm, cos_hbm, sin_hbm, out_hbm,
                    idx_v, rows_v, out_v, cos_v, sin_v, gsem):
    wid = lax.axis_index("s") * 2 + lax.axis_index("c")
    pltpu.sync_copy(cos_hbm, cos_v)
    pltpu.sync_copy(sin_hbm, sin_v)
    base = wid * SEQ_PER_W

    def seq_body(i, carry):
        b = base + i
        pltpu.sync_copy(x_hbm.at[b], idx_v)
        cps = [pltpu.async_copy(table_hbm.at[idx_v.at[c]], rows_v.at[c], gsem)
               for c in range(NCH)]
        for cp in cps:
            cp.wait()

        def row_body(rr, inner):
            for c in range(NCH):
                pos = c * CH + rr
                for j in range(HALF // 16):
                    e = rows_v[c, rr, pl.ds(j * 16, 16)]
                    o = rows_v[c, rr, pl.ds(HALF + j * 16, 16)]
                    cv = cos_v[pos, pl.ds(j * 16, 16)]
                    sv = sin_v[pos, pl.ds(j * 16, 16)]
                    out_v[c, rr, pl.ds(j * 16, 16)] = e * cv - o * sv
                    out_v[c, rr, pl.ds(HALF + j * 16, 16)] = e * sv + o * cv
            return inner

        lax.fori_loop(0, CH, row_body, 0)
        pltpu.sync_copy(out_v, out_hbm.at[b])
        return carry

    lax.fori_loop(0, SEQ_PER_W, seq_body, 0)


def kernel(x, table):
    x = x.reshape(B, NCH, CH).astype(jnp.int32)
    table = table.astype(jnp.float32)
    cos_t, sin_t = _make_tables()
    out = _sc_rope_gather(x, table, cos_t, sin_t)
    return out.reshape(B, L, D)


# speedup vs baseline: 4.3806x; 1.4889x over previous
import functools
import math

import jax
import jax.numpy as jnp
from jax import lax
from jax.experimental import pallas as pl
from jax.experimental.pallas import tpu as pltpu
from jax.experimental.pallas import tpu_sc as plsc

B, L, D, VOCAB = 1024, 200, 128, 100000
HALF = D // 2
CH = 100
NCHUNK = B * L // CH
NW = 32
CPW = NCHUNK // NW
NBUF = 4
DEPTH = 3


def _trig_body(cos_ref, sin_ref):
    pos = lax.broadcasted_iota(jnp.int32, (L, HALF), 0).astype(jnp.float32)
    fi = lax.broadcasted_iota(jnp.int32, (L, HALF), 1).astype(jnp.float32)
    ang = pos * jnp.exp(fi * (-math.log(10000.0) / D))
    cos_ref[...] = jnp.cos(ang)
    sin_ref[...] = jnp.sin(ang)


def _make_tables():
    return pl.pallas_call(
        _trig_body,
        out_shape=(
            jax.ShapeDtypeStruct((L, HALF), jnp.float32),
            jax.ShapeDtypeStruct((L, HALF), jnp.float32),
        ),
    )()


@functools.partial(
    pl.kernel,
    mesh=plsc.VectorSubcoreMesh(core_axis_name="c", subcore_axis_name="s"),
    out_type=jax.ShapeDtypeStruct((NCHUNK, CH, D), jnp.float32),
    scratch_types=[
        pltpu.VMEM((NBUF, CH), jnp.int32),
        pltpu.VMEM((NBUF, CH, D), jnp.float32),
        pltpu.VMEM((L, HALF), jnp.float32),
        pltpu.VMEM((L, HALF), jnp.float32),
        pltpu.SemaphoreType.DMA,
        pltpu.SemaphoreType.DMA,
    ],
)
def _sc_rope_gather(x_hbm, table_hbm, cos_hbm, sin_hbm, out_hbm,
                    idx_v, rows_v, cos_v, sin_v, gsem, osem):
    wid = lax.axis_index("s") * 2 + lax.axis_index("c")
    pltpu.sync_copy(cos_hbm, cos_v)
    pltpu.sync_copy(sin_hbm, sin_v)
    base = wid * CPW

    def fire_gather(k, slot):
        pltpu.sync_copy(x_hbm.at[base + k], idx_v.at[slot])
        pltpu.make_async_copy(
            table_hbm.at[idx_v.at[slot]], rows_v.at[slot], gsem).start()

    def wait_gather(slot):
        pltpu.make_async_copy(
            table_hbm.at[idx_v.at[slot]], rows_v.at[slot], gsem).wait()

    def fire_out(k, slot):
        pltpu.make_async_copy(
            rows_v.at[slot], out_hbm.at[base + k], osem).start()

    def wait_out(k, slot):
        pltpu.make_async_copy(
            rows_v.at[slot], out_hbm.at[base + k], osem).wait()

    def compute(k, p):
        pbase = lax.rem(k, 2) * CH

        def row_body(rr, inner):
            pos = pbase + rr
            for j in range(HALF // 16):
                e = rows_v[p, rr, pl.ds(j * 16, 16)]
                o = rows_v[p, rr, pl.ds(HALF + j * 16, 16)]
                cv = cos_v[pos, pl.ds(j * 16, 16)]
                sv = sin_v[pos, pl.ds(j * 16, 16)]
                rows_v[p, rr, pl.ds(j * 16, 16)] = e * cv - o * sv
                rows_v[p, rr, pl.ds(HALF + j * 16, 16)] = e * sv + o * cv
            return inner

        lax.fori_loop(0, CH, row_body, 0)

    def step(k, p, first, last):
        wait_gather(p)
        compute(k, p)
        fire_out(k, p)
        if not first:
            wait_out(k - 1, (p + DEPTH) % NBUF)
        if not last:
            fire_gather(k + DEPTH, (p + DEPTH) % NBUF)

    for s in range(DEPTH):
        fire_gather(s, s)

    HEAD = NBUF
    TAIL = NBUF
    NGROUPS = (CPW - HEAD - TAIL) // NBUF

    for k in range(HEAD):
        step(k, k % NBUF, first=(k == 0), last=(k + DEPTH >= CPW))

    def group_body(g, carry):
        for p in range(NBUF):
            k = HEAD + g * NBUF + p
            step(k, (HEAD + p) % NBUF, first=False, last=False)
        return carry

    lax.fori_loop(0, NGROUPS, group_body, 0)

    for k in range(CPW - TAIL, CPW):
        step(k, k % NBUF, first=False, last=(k + DEPTH >= CPW))

    wait_out(CPW - 1, (CPW - 1) % NBUF)


def kernel(x, table):
    x = x.reshape(NCHUNK, CH).astype(jnp.int32)
    table = table.astype(jnp.float32)
    cos_t, sin_t = _make_tables()
    out = _sc_rope_gather(x, table, cos_t, sin_t)
    return out.reshape(B, L, D)
